# Initial kernel scaffold; baseline (speedup 1.0000x reference)
import jax
import jax.numpy as jnp
from jax.experimental import pallas as pl


def kernel(x, edge_index, W, b):
    raise NotImplementedError("write your pallas kernel here")


# speedup vs baseline: 9.4411x; 9.4411x over previous
import functools

import jax
import jax.numpy as jnp
from jax import lax
from jax.experimental import pallas as pl
from jax.experimental.pallas import tpu as pltpu
from jax.experimental.pallas import tpu_sc as plsc

NC = 2
NS = 16
NW = NC * NS
LANES = 16
CHUNK = 128


def _deg_kernel(n_groups, group, n_pad, d):
    mesh = plsc.VectorSubcoreMesh(
        core_axis_name="c", subcore_axis_name="s",
        num_cores=NC, num_subcores=NS)
    rpt = n_pad // NS
    n_zero = rpt // CHUNK

    @functools.partial(
        pl.kernel,
        out_type=jax.ShapeDtypeStruct((NC * n_pad, d), jnp.float32),
        mesh=mesh,
        scratch_types=[
            pltpu.VMEM((group, CHUNK), jnp.int32),
            pltpu.VMEM((CHUNK, d), jnp.float32),
            pltpu.VMEM_SHARED((n_pad, d), jnp.float32),
            pltpu.SemaphoreType.DMA,
        ],
    )
    def k(dst4, ones_h, zero_h, degp, dst_idx, buf_v, acc, sem):
        c = lax.axis_index("c")
        s = lax.axis_index("s")
        wid = s * NC + c

        pltpu.sync_copy(zero_h, buf_v)
        for m in range(n_zero):
            pltpu.sync_copy(buf_v, acc.at[pl.ds(s * rpt + m * CHUNK, CHUNK)])
        plsc.subcore_barrier()
        pltpu.sync_copy(ones_h, buf_v)

        def body(t, _):
            pltpu.sync_copy(dst4.at[wid, t], dst_idx)
            descs = [
                pltpu.async_copy(buf_v, acc.at[dst_idx.at[u]], sem, add=True)
                for u in range(group)]
            for dsc in descs:
                dsc.wait()
            return 0
        lax.fori_loop(0, n_groups, body, 0)
        plsc.subcore_barrier()
        off = pl.multiple_of(c * n_pad + s * rpt, CHUNK)
        pltpu.sync_copy(acc.at[pl.ds(s * rpt, rpt)],
                        degp.at[pl.ds(off, rpt)])

    return k


def _agg_kernel(n_groups, group, n_pad, d):
    mesh = plsc.VectorSubcoreMesh(
        core_axis_name="c", subcore_axis_name="s",
        num_cores=NC, num_subcores=NS)
    rpt = n_pad // NS
    n_zero = rpt // CHUNK
    assert group % 2 == 0

    @functools.partial(
        pl.kernel,
        out_type=jax.ShapeDtypeStruct((NC * n_pad, d), jnp.float32),
        mesh=mesh,
        scratch_types=[
            pltpu.VMEM((group, CHUNK), jnp.int32),
            pltpu.VMEM((group, CHUNK), jnp.int32),
            pltpu.VMEM((CHUNK, d), jnp.float32),
            pltpu.VMEM((CHUNK, d), jnp.float32),
            pltpu.VMEM_SHARED((n_pad, d), jnp.float32),
            pltpu.SemaphoreType.DMA,
            pltpu.SemaphoreType.DMA,
        ],
    )
    def k(src4, dst4, g, q, src_idx, dst_idx, rows0, rows1, acc, sem0, sem1):
        c = lax.axis_index("c")
        s = lax.axis_index("s")
        wid = s * NC + c

        def zfill(i, _):
            for t in range(d // LANES):
                rows0[i, pl.ds(t * LANES, LANES)] = jnp.zeros(
                    (LANES,), jnp.float32)
            return 0
        lax.fori_loop(0, CHUNK, zfill, 0)
        for m in range(n_zero):
            pltpu.sync_copy(rows0, acc.at[pl.ds(s * rpt + m * CHUNK, CHUNK)])
        plsc.subcore_barrier()

        def body(t, _):
            pltpu.sync_copy(src4.at[wid, t], src_idx)
            pltpu.sync_copy(dst4.at[wid, t], dst_idx)
            for u in range(0, group, 2):
                d0 = pltpu.async_copy(g.at[src_idx.at[u]], rows0, sem0)
                d1 = pltpu.async_copy(g.at[src_idx.at[u + 1]], rows1, sem1)
                d0.wait()
                pltpu.sync_copy(rows0, acc.at[dst_idx.at[u]], add=True)
                d1.wait()
                pltpu.sync_copy(rows1, acc.at[dst_idx.at[u + 1]], add=True)
            return 0
        lax.fori_loop(0, n_groups, body, 0)
        plsc.subcore_barrier()
        off = pl.multiple_of(c * n_pad + s * rpt, CHUNK)
        pltpu.sync_copy(acc.at[pl.ds(s * rpt, rpt)],
                        q.at[pl.ds(off, rpt)])

    return k


def _dense_body(x_ref, w_ref, degp_ref, g_ref):
    deg = degp_ref[0, :, 0:1] + degp_ref[1, :, 0:1]
    dis = lax.rsqrt(jnp.maximum(deg, 1e-12))
    h = jnp.dot(x_ref[...], w_ref[...], preferred_element_type=jnp.float32)
    g_ref[...] = h * dis


def _finish_body(q_ref, degp_ref, b_ref, o_ref):
    deg = degp_ref[0, :, 0:1] + degp_ref[1, :, 0:1]
    dis = lax.rsqrt(jnp.maximum(deg, 1e-12))
    o_ref[...] = (q_ref[0] + q_ref[1]) * dis + b_ref[...]


def kernel(x, edge_index, W, b):
    n, d_in = x.shape
    d_out = W.shape[1]
    e = edge_index.shape[1]

    src = edge_index[0].astype(jnp.int32)
    dst = edge_index[1].astype(jnp.int32)
    loop = jnp.arange(n, dtype=jnp.int32)
    src = jnp.concatenate([src, loop])
    dst = jnp.concatenate([dst, loop])

    t = e + n
    unit = NW * CHUNK
    group = 6
    n_groups = -(-t // (unit * group))
    n_chunks = n_groups * group
    t_pad = n_chunks * unit
    src = jnp.concatenate([src, jnp.zeros((t_pad - t,), jnp.int32)])
    dst = jnp.concatenate([dst, jnp.full((t_pad - t,), n, jnp.int32)])
    src4 = src.reshape(NW, n_groups, group, CHUNK)
    dst4 = dst.reshape(NW, n_groups, group, CHUNK)

    stripe = NS * CHUNK
    n_pad = -(-(n + 1) // stripe) * stripe

    degp = _deg_kernel(n_groups, group, n_pad, d_out)(
        dst4, jnp.ones((CHUNK, d_out), jnp.float32),
        jnp.zeros((CHUNK, d_out), jnp.float32))
    degp3 = degp.reshape(NC, n_pad, d_out)

    blk = 1024
    grid = (n_pad // blk,)
    xp = jnp.pad(x, ((0, n_pad - n), (0, 0)))
    g = pl.pallas_call(
        _dense_body,
        grid=grid,
        in_specs=[
            pl.BlockSpec((blk, d_in), lambda i: (i, 0)),
            pl.BlockSpec((d_in, d_out), lambda i: (0, 0)),
            pl.BlockSpec((NC, blk, d_out), lambda i: (0, i, 0)),
        ],
        out_specs=pl.BlockSpec((blk, d_out), lambda i: (i, 0)),
        out_shape=jax.ShapeDtypeStruct((n_pad, d_out), jnp.float32),
    )(xp, W, degp3)

    q = _agg_kernel(n_groups, group, n_pad, d_out)(src4, dst4, g)
    q3 = q.reshape(NC, n_pad, d_out)

    out = pl.pallas_call(
        _finish_body,
        grid=grid,
        in_specs=[
            pl.BlockSpec((NC, blk, d_out), lambda i: (0, i, 0)),
            pl.BlockSpec((NC, blk, d_out), lambda i: (0, i, 0)),
            pl.BlockSpec((1, d_out), lambda i: (0, 0)),
        ],
        out_specs=pl.BlockSpec((blk, d_out), lambda i: (i, 0)),
        out_shape=jax.ShapeDtypeStruct((n_pad, d_out), jnp.float32),
    )(q3, degp3, b.reshape(1, d_out))

    return out[:n]


# speedup vs baseline: 30.0235x; 3.1801x over previous
import functools

import jax
import jax.numpy as jnp
from jax import lax
from jax.experimental import pallas as pl
from jax.experimental.pallas import tpu as pltpu
from jax.experimental.pallas import tpu_sc as plsc

NC = 2
NS = 16
NW = NC * NS
LANES = 16
CHUNK = 128


def _deg_kernel(n_groups, group, n_pad, d):
    mesh = plsc.VectorSubcoreMesh(
        core_axis_name="c", subcore_axis_name="s",
        num_cores=NC, num_subcores=NS)
    rpt = n_pad // NS
    n_zero = rpt // CHUNK

    @functools.partial(
        pl.kernel,
        out_type=jax.ShapeDtypeStruct((NC * n_pad, d), jnp.float32),
        mesh=mesh,
        scratch_types=[
            pltpu.VMEM((group, CHUNK), jnp.int32),
            pltpu.VMEM((CHUNK, d), jnp.float32),
            pltpu.VMEM_SHARED((n_pad, d), jnp.float32),
            pltpu.SemaphoreType.DMA,
        ],
    )
    def k(dst4, ones_h, zero_h, degp, dst_idx, buf_v, acc, sem):
        c = lax.axis_index("c")
        s = lax.axis_index("s")
        wid = s * NC + c

        pltpu.sync_copy(zero_h, buf_v)
        for m in range(n_zero):
            pltpu.sync_copy(buf_v, acc.at[pl.ds(s * rpt + m * CHUNK, CHUNK)])
        plsc.subcore_barrier()
        pltpu.sync_copy(ones_h, buf_v)

        def body(t, _):
            pltpu.sync_copy(dst4.at[wid, t], dst_idx)
            descs = [
                pltpu.async_copy(buf_v, acc.at[dst_idx.at[u]], sem, add=True)
                for u in range(group)]
            for dsc in descs:
                dsc.wait()
            return 0
        lax.fori_loop(0, n_groups, body, 0)
        plsc.subcore_barrier()
        off = pl.multiple_of(c * n_pad + s * rpt, CHUNK)
        pltpu.sync_copy(acc.at[pl.ds(s * rpt, rpt)],
                        degp.at[pl.ds(off, rpt)])

    return k


def _agg_kernel(n_groups, group, n_pad, d):
    mesh = plsc.VectorSubcoreMesh(
        core_axis_name="c", subcore_axis_name="s",
        num_cores=NC, num_subcores=NS)
    rpt = n_pad // NS
    n_zero = rpt // CHUNK
    assert group % 2 == 0

    @functools.partial(
        pl.kernel,
        out_type=jax.ShapeDtypeStruct((NC * n_pad, d), jnp.float32),
        mesh=mesh,
        scratch_types=[
            pltpu.VMEM((group, CHUNK), jnp.int32),
            pltpu.VMEM((group, CHUNK), jnp.int32),
            pltpu.VMEM((CHUNK, d), jnp.float32),
            pltpu.VMEM((CHUNK, d), jnp.float32),
            pltpu.VMEM_SHARED((n_pad, d), jnp.float32),
            pltpu.SemaphoreType.DMA,
            pltpu.SemaphoreType.DMA,
        ],
    )
    def k(src4, dst4, g, q, src_idx, dst_idx, rows0, rows1, acc, sem0, sem1):
        c = lax.axis_index("c")
        s = lax.axis_index("s")
        wid = s * NC + c

        def zfill(i, _):
            for t in range(d // LANES):
                rows0[i, pl.ds(t * LANES, LANES)] = jnp.zeros(
                    (LANES,), jnp.float32)
            return 0
        lax.fori_loop(0, CHUNK, zfill, 0)
        for m in range(n_zero):
            pltpu.sync_copy(rows0, acc.at[pl.ds(s * rpt + m * CHUNK, CHUNK)])
        plsc.subcore_barrier()

        def body(t, _):
            pltpu.sync_copy(src4.at[wid, t], src_idx)
            pltpu.sync_copy(dst4.at[wid, t], dst_idx)
            for u in range(0, group, 2):
                d0 = pltpu.async_copy(g.at[src_idx.at[u]], rows0, sem0)
                d1 = pltpu.async_copy(g.at[src_idx.at[u + 1]], rows1, sem1)
                d0.wait()
                pltpu.sync_copy(rows0, acc.at[dst_idx.at[u]], add=True)
                d1.wait()
                pltpu.sync_copy(rows1, acc.at[dst_idx.at[u + 1]], add=True)
            return 0
        lax.fori_loop(0, n_groups, body, 0)
        plsc.subcore_barrier()
        off = pl.multiple_of(c * n_pad + s * rpt, CHUNK)
        pltpu.sync_copy(acc.at[pl.ds(s * rpt, rpt)],
                        q.at[pl.ds(off, rpt)])

    return k


def _agg_kernel2(n_groups, group, n_pad, d, ce):
    mesh = plsc.VectorSubcoreMesh(
        core_axis_name="c", subcore_axis_name="s",
        num_cores=NC, num_subcores=NS)
    rpt = n_pad // NS
    n_zero = rpt // ce
    nbuf = 4
    assert group > nbuf

    @functools.partial(
        pl.kernel,
        out_type=jax.ShapeDtypeStruct((NC * n_pad, d), jnp.float32),
        mesh=mesh,
        scratch_types=[
            pltpu.VMEM((group, ce), jnp.int32),
            pltpu.VMEM((group, ce), jnp.int32),
            pltpu.VMEM((ce, d), jnp.float32),
            pltpu.VMEM((ce, d), jnp.float32),
            pltpu.VMEM((ce, d), jnp.float32),
            pltpu.VMEM((ce, d), jnp.float32),
            pltpu.VMEM_SHARED((n_pad, d), jnp.float32),
            pltpu.SemaphoreType.DMA, pltpu.SemaphoreType.DMA,
            pltpu.SemaphoreType.DMA, pltpu.SemaphoreType.DMA,
            pltpu.SemaphoreType.DMA, pltpu.SemaphoreType.DMA,
            pltpu.SemaphoreType.DMA, pltpu.SemaphoreType.DMA,
        ],
    )
    def k(src4, dst4, g, q, src_idx, dst_idx, r0, r1, r2, r3, acc,
          g0, g1, g2, g3, s0, s1, s2, s3):
        rows = [r0, r1, r2, r3]
        gsem = [g0, g1, g2, g3]
        ssem = [s0, s1, s2, s3]
        c = lax.axis_index("c")
        s = lax.axis_index("s")
        wid = s * NC + c

        def zfill(i, _):
            for t in range(d // LANES):
                r0[i, pl.ds(t * LANES, LANES)] = jnp.zeros(
                    (LANES,), jnp.float32)
            return 0
        lax.fori_loop(0, ce, zfill, 0)
        for m in range(n_zero):
            pltpu.sync_copy(r0, acc.at[pl.ds(s * rpt + m * ce, ce)])
        plsc.subcore_barrier()

        def body(t, _):
            pltpu.sync_copy(src4.at[wid, t], src_idx)
            pltpu.sync_copy(dst4.at[wid, t], dst_idx)
            gd = [None] * group
            sd = [None] * group
            for u in range(nbuf):
                gd[u] = pltpu.async_copy(
                    g.at[src_idx.at[u]], rows[u], gsem[u])
            for u in range(group):
                gd[u].wait()
                sd[u] = pltpu.async_copy(
                    rows[u % nbuf], acc.at[dst_idx.at[u]], ssem[u % nbuf],
                    add=True)
                v = u - 2
                if v >= 0 and v + nbuf < group:
                    sd[v].wait()
                    gd[v + nbuf] = pltpu.async_copy(
                        g.at[src_idx.at[v + nbuf]], rows[v % nbuf],
                        gsem[v % nbuf])
            for u in range(max(0, group - nbuf), group):
                sd[u].wait()
            return 0
        lax.fori_loop(0, n_groups, body, 0)
        plsc.subcore_barrier()
        off = pl.multiple_of(c * n_pad + s * rpt, CHUNK)
        pltpu.sync_copy(acc.at[pl.ds(s * rpt, rpt)],
                        q.at[pl.ds(off, rpt)])

    return k


def _dense_body(x_ref, w_ref, degp_ref, g_ref):
    deg = degp_ref[0, :, 0:1] + degp_ref[1, :, 0:1]
    dis = lax.rsqrt(jnp.maximum(deg, 1e-12))
    h = jnp.dot(x_ref[...], w_ref[...], preferred_element_type=jnp.float32)
    g_ref[...] = h * dis


def _finish_body(q_ref, degp_ref, b_ref, o_ref):
    deg = degp_ref[0, :, 0:1] + degp_ref[1, :, 0:1]
    dis = lax.rsqrt(jnp.maximum(deg, 1e-12))
    o_ref[...] = (q_ref[0] + q_ref[1]) * dis + b_ref[...]


def kernel(x, edge_index, W, b):
    n, d_in = x.shape
    d_out = W.shape[1]
    e = edge_index.shape[1]

    src = edge_index[0].astype(jnp.int32)
    dst = edge_index[1].astype(jnp.int32)
    loop = jnp.arange(n, dtype=jnp.int32)
    src = jnp.concatenate([src, loop])
    dst = jnp.concatenate([dst, loop])

    stripe = NS * CHUNK
    n_pad = -(-(n + 1) // stripe) * stripe

    t = e + n
    group = 6
    ce = 64
    group_a = 12
    unit = NW * CHUNK * group
    n_groups = -(-t // unit)
    t_pad = n_groups * unit
    fill = n + jnp.arange(t_pad - t, dtype=jnp.int32) % (n_pad - n)
    src = jnp.concatenate([src, fill])
    dst = jnp.concatenate([dst, fill])
    dst4d = dst.reshape(NW, n_groups, group, CHUNK)
    src4a = src.reshape(NW, n_groups, group_a, ce)
    dst4a = dst.reshape(NW, n_groups, group_a, ce)

    degw = 32
    degp = _deg_kernel(n_groups, group, n_pad, degw)(
        dst4d, jnp.ones((CHUNK, degw), jnp.float32),
        jnp.zeros((CHUNK, degw), jnp.float32))
    degp3 = degp.reshape(NC, n_pad, degw)

    blk = 1024
    grid = (n_pad // blk,)
    xp = jnp.pad(x, ((0, n_pad - n), (0, 0)))
    g = pl.pallas_call(
        _dense_body,
        grid=grid,
        in_specs=[
            pl.BlockSpec((blk, d_in), lambda i: (i, 0)),
            pl.BlockSpec((d_in, d_out), lambda i: (0, 0)),
            pl.BlockSpec((NC, blk, degw), lambda i: (0, i, 0)),
        ],
        out_specs=pl.BlockSpec((blk, d_out), lambda i: (i, 0)),
        out_shape=jax.ShapeDtypeStruct((n_pad, d_out), jnp.float32),
    )(xp, W, degp3)

    q = _agg_kernel2(n_groups, group_a, n_pad, d_out, ce)(src4a, dst4a, g)
    q3 = q.reshape(NC, n_pad, d_out)

    out = pl.pallas_call(
        _finish_body,
        grid=grid,
        in_specs=[
            pl.BlockSpec((NC, blk, d_out), lambda i: (0, i, 0)),
            pl.BlockSpec((NC, blk, degw), lambda i: (0, i, 0)),
            pl.BlockSpec((1, d_out), lambda i: (0, 0)),
        ],
        out_specs=pl.BlockSpec((blk, d_out), lambda i: (i, 0)),
        out_shape=jax.ShapeDtypeStruct((n_pad, d_out), jnp.float32),
    )(q3, degp3, b.reshape(1, d_out))

    return out[:n]
